# baseline (speedup 1.0000x reference)
import jax
import jax.numpy as jnp
from jax.experimental import pallas as pl


BLOCK_R = 64


def _add_kernel(x_ref, t_ref, o_ref):
    o_ref[...] = x_ref[...] + t_ref[...]


def kernel(inputs, table):
    b, f, d = inputs.shape
    x2 = jnp.transpose(inputs, (1, 2, 0)).reshape(f * d, b)
    t2 = table.reshape(f * d, 1)

    out2 = pl.pallas_call(
        _add_kernel,
        grid=(f * d // BLOCK_R,),
        in_specs=[
            pl.BlockSpec((BLOCK_R, b), lambda i: (i, 0)),
            pl.BlockSpec((BLOCK_R, 1), lambda i: (i, 0)),
        ],
        out_specs=pl.BlockSpec((BLOCK_R, b), lambda i: (i, 0)),
        out_shape=jax.ShapeDtypeStruct((f * d, b), inputs.dtype),
    )(x2, t2)
    return jnp.transpose(out2.reshape(f, d, b), (2, 0, 1))


# speedup vs baseline: 1.0140x; 1.0140x over previous
import jax
import jax.numpy as jnp
from jax.experimental import pallas as pl


BLOCK_R = 200


def _add_kernel(x_ref, t_ref, o_ref):
    o_ref[...] = x_ref[...] + t_ref[...]


def kernel(inputs, table):
    b, f, d = inputs.shape
    x2 = jnp.transpose(inputs, (1, 2, 0)).reshape(f * d, b)
    t2 = table.reshape(f * d, 1)

    out2 = pl.pallas_call(
        _add_kernel,
        grid=(f * d // BLOCK_R,),
        in_specs=[
            pl.BlockSpec((BLOCK_R, b), lambda i: (i, 0)),
            pl.BlockSpec((BLOCK_R, 1), lambda i: (i, 0)),
        ],
        out_specs=pl.BlockSpec((BLOCK_R, b), lambda i: (i, 0)),
        out_shape=jax.ShapeDtypeStruct((f * d, b), inputs.dtype),
    )(x2, t2)
    return jnp.transpose(out2.reshape(f, d, b), (2, 0, 1))


# speedup vs baseline: 1.0367x; 1.0224x over previous
import jax
import jax.numpy as jnp
from jax.experimental import pallas as pl


BLOCK_F = 5


def _add_kernel(x_ref, t_ref, o_ref):
    i = pl.program_id(0)
    t_blk = t_ref[pl.ds(i * BLOCK_F, BLOCK_F), :]
    o_ref[...] = x_ref[...] + t_blk[:, :, None]


def kernel(inputs, table):
    b, f, d = inputs.shape
    x3 = jnp.transpose(inputs, (1, 2, 0))

    out3 = pl.pallas_call(
        _add_kernel,
        grid=(f // BLOCK_F,),
        in_specs=[
            pl.BlockSpec((BLOCK_F, d, b), lambda i: (i, 0, 0)),
            pl.BlockSpec((f, d), lambda i: (0, 0)),
        ],
        out_specs=pl.BlockSpec((BLOCK_F, d, b), lambda i: (i, 0, 0)),
        out_shape=jax.ShapeDtypeStruct((f, d, b), inputs.dtype),
    )(x3, table)
    return jnp.transpose(out3, (2, 0, 1))
